# Initial kernel scaffold; baseline (speedup 1.0000x reference)
import jax
import jax.numpy as jnp
from jax.experimental import pallas as pl


def kernel(x, layer_idx):
    raise NotImplementedError("write your pallas kernel here")


# speedup vs baseline: 2.6941x; 2.6941x over previous
import jax
import jax.numpy as jnp
from jax.experimental import pallas as pl
from jax.experimental.pallas import tpu as pltpu

N = 32768
TOPK = 16384
LOGN = 15
CB = 128
CH = 256


def _pair_chunk(buf, rowa, rowb, desc, ch):
    a = buf[pl.ds(rowa, ch), :]
    b = buf[pl.ds(rowb, ch), :]
    hi = jnp.maximum(a, b)
    lo = jnp.minimum(a, b)
    if desc is None:
        buf[pl.ds(rowa, ch), :] = hi
        buf[pl.ds(rowb, ch), :] = lo
    else:
        buf[pl.ds(rowa, ch), :] = jnp.where(desc, hi, lo)
        buf[pl.ds(rowb, ch), :] = jnp.where(desc, lo, hi)


def _substage_big(buf, p, j, nr):
    s = 1 << j
    n_iter = (nr // 2) // CH

    def body(t, carry):
        tb = t * CH
        off = tb & (s - 1)
        rowa = ((tb >> j) << (j + 1)) | off
        if p is None:
            desc = None
        else:
            desc = ((rowa >> p) & 1) == 0
        _pair_chunk(buf, rowa, rowa + s, desc, CH)
        return carry

    jax.lax.fori_loop(0, n_iter, body, 0)


def _substage_small(buf, p, j, nr):
    s = 1 << j
    n_iter = nr // CH
    m_per_chunk = CH // (2 * s)

    def body(t, carry):
        r0 = t * CH
        c = buf[pl.ds(r0, CH), :].reshape(m_per_chunk, 2, s, CB)
        a = c[:, 0]
        b = c[:, 1]
        hi = jnp.maximum(a, b)
        lo = jnp.minimum(a, b)
        if p is None:
            na, nb = hi, lo
        else:
            shift = p - j - 1
            it = jax.lax.broadcasted_iota(jnp.int32, (m_per_chunk, 1, 1), 0)
            mg = it + (r0 >> (j + 1))
            desc = ((mg >> shift) & 1) == 0
            na = jnp.where(desc, hi, lo)
            nb = jnp.where(desc, lo, hi)
        buf[pl.ds(r0, CH), :] = jnp.concatenate(
            [na[:, None], nb[:, None]], axis=1).reshape(CH, CB)
        return carry

    jax.lax.fori_loop(0, n_iter, body, 0)


def _substage(buf, p, j, nr):
    if (1 << j) >= CH:
        _substage_big(buf, p, j, nr)
    else:
        _substage_small(buf, p, j, nr)


def _half_merge(buf):
    def body(t, carry):
        r0 = t * CH
        a = buf[pl.ds(r0, CH), :]
        b = buf[pl.ds(r0 + TOPK, CH), :]
        buf[pl.ds(r0, CH), :] = jnp.maximum(a, b)
        return carry

    jax.lax.fori_loop(0, TOPK // CH, body, 0)


def _sort_body(x_hbm, o_hbm, buf, sem_in, sem_out):
    i = pl.program_id(0)
    cin = pltpu.make_async_copy(
        x_hbm.at[:, pl.ds(i * CB, CB)], buf, sem_in)
    cin.start()
    cin.wait()

    for p in range(1, LOGN):
        for j in range(p - 1, -1, -1):
            _substage(buf, p, j, N)
    _half_merge(buf)
    for j in range(LOGN - 2, -1, -1):
        _substage(buf, None, j, TOPK)

    cout = pltpu.make_async_copy(
        buf.at[pl.ds(0, TOPK), :], o_hbm.at[:, pl.ds(i * CB, CB)], sem_out)
    cout.start()
    cout.wait()


def _topk_columns(xt):
    n, r = xt.shape
    return pl.pallas_call(
        _sort_body,
        grid=(r // CB,),
        in_specs=[pl.BlockSpec(memory_space=pl.ANY)],
        out_specs=pl.BlockSpec(memory_space=pl.ANY),
        out_shape=jax.ShapeDtypeStruct((n // 2, r), jnp.float32),
        scratch_shapes=[
            pltpu.VMEM((N, CB), jnp.float32),
            pltpu.SemaphoreType.DMA,
            pltpu.SemaphoreType.DMA,
        ],
        compiler_params=pltpu.CompilerParams(
            dimension_semantics=("arbitrary",),
        ),
    )(xt)


@jax.jit
def kernel(x, layer_idx):
    b, ch, n = x.shape
    xt = x.reshape(b * ch, n).T
    out_t = _topk_columns(xt)
    out = out_t.T.reshape(b, ch, n // 2)
    return out + jnp.zeros((), dtype=out.dtype) * layer_idx
